# baseline (speedup 1.0000x reference)
import functools

import jax
import jax.numpy as jnp
from jax.experimental import pallas as pl
from jax.experimental.pallas import tpu as pltpu

_BIG = 1e9


def _sum11(x):
    return jnp.sum(jnp.sum(x, axis=0, keepdims=True), axis=1, keepdims=True)


def _sigmoids(d):
    ed = jnp.exp(-jnp.abs(d))
    r = 1.0 / (1.0 + ed)
    pos = d >= 0.0
    s1 = jnp.where(pos, r, ed * r)
    s0 = jnp.where(pos, ed * r, r)
    return s1, s0


def _loss_kernel(anch_ref, bc_ref, gt_ref, out_t_ref, out_cl_ref, out_co_ref,
                 sacc, bv, bidx, bl1, bd, aux,
                 *, n_img, n_chunks, chunk, n_total, n_gt):
    i = pl.program_id(0)
    k = pl.program_id(1)

    @pl.when(jnp.logical_and(i == 0, k == 0))
    def _():
        sacc[3:5, :] = jnp.zeros((2, 128), jnp.float32)

    @pl.when(k == 0)
    def _():
        sacc[0:3, :] = jnp.zeros((3, 128), jnp.float32)
        bv[...] = jnp.full((n_gt, 128), -1.0, jnp.float32)

    a_x0 = anch_ref[0:1, :]
    a_y0 = anch_ref[1:2, :]
    a_x1 = anch_ref[2:3, :]
    a_y1 = anch_ref[3:4, :]
    area_a = (a_x1 - a_x0) * (a_y1 - a_y0)

    b0 = bc_ref[0, 0:1, :]
    b1 = bc_ref[0, 1:2, :]
    b2 = bc_ref[0, 2:3, :]
    b3 = bc_ref[0, 3:4, :]
    d = bc_ref[0, 5:6, :] - bc_ref[0, 4:5, :]

    gtb = gt_ref[0]
    g_x0 = gtb[:, 0:1]
    g_y0 = gtb[:, 1:2]
    g_x1 = g_x0 + gtb[:, 2:3]
    g_y1 = g_y0 + gtb[:, 3:4]
    validv = gtb[:, 4:5] > 0.5
    area_g = (g_x1 - g_x0) * (g_y1 - g_y0)

    wx = jnp.maximum(jnp.minimum(a_x1, g_x1) - jnp.maximum(a_x0, g_x0), 0.0)
    wy = jnp.maximum(jnp.minimum(a_y1, g_y1) - jnp.maximum(a_y0, g_y0), 0.0)
    inter = wx * wy
    union = area_a + area_g - inter
    iou = inter / union
    maskf = jnp.where(iou > 0.5, 1.0, 0.0)

    l1 = (jnp.abs(b0 - g_x0) + jnp.abs(b1 - g_y0)
          + jnp.abs(b2 - g_x1) + jnp.abs(b3 - g_y1))

    sacc[1:2, 0:1] += _sum11(maskf * l1)
    cnt_row = jnp.sum(maskf, axis=0, keepdims=True)
    sacc[2:3, 0:1] += jnp.sum(cnt_row, axis=1, keepdims=True)

    anyrow = cnt_row > 0.5
    s1, s0 = _sigmoids(d)
    fl_pos = (-5.0) * s0 * s0 * jnp.log(s1)
    fl_neg = (-1.0) * s1 * s1 * jnp.log(s0)
    fl = jnp.where(anyrow, fl_pos, fl_neg)
    sacc[0:1, 0:1] += _sum11(fl)

    cmax = jnp.max(iou, axis=1, keepdims=True)
    lane = jax.lax.broadcasted_iota(jnp.int32, iou.shape, 1)
    cand_lane = jnp.argmax(iou, axis=1).astype(jnp.int32)[:, None]
    onehot = jnp.where(lane == cand_lane, 1.0, 0.0)
    ext_row = (jnp.clip(d, -1000.0, 1000.0)
               + jnp.where(anyrow, 65536.0, 0.0))
    cand_l1 = jnp.sum(onehot * l1, axis=1, keepdims=True)
    cand_e = jnp.sum(onehot * ext_row, axis=1, keepdims=True)
    cand_idx = (cand_lane + chunk * k).astype(jnp.float32)

    upd = cmax > bv[:, 0:1]
    bv[...] = jnp.where(upd, cmax, bv[...])
    bidx[...] = jnp.where(upd, cand_idx, bidx[...])
    bl1[...] = jnp.where(upd, cand_l1, bl1[...])
    bd[...] = jnp.where(upd, cand_e, bd[...])

    @pl.when(k == n_chunks - 1)
    def _():
        bv_v = bv[:, 0:1]
        bidx_v = bidx[:, 0:1]
        validf = jnp.where(validv, 1.0, 0.0)

        add_pair = jnp.where(jnp.logical_and(validv, bv_v <= 0.5), 1.0, 0.0)
        add_l1 = _sum11(add_pair * bl1[:, 0:1])
        add_cnt = _sum11(add_pair)
        csum_t = sacc[1:2, 0:1] + add_l1
        cnt_t = (sacc[2:3, 0:1] + add_cnt) * 4.0
        coord_img = csum_t / cnt_t

        ev = bd[:, 0:1]
        tentv = ev > 32768.0
        dv = ev - jnp.where(tentv, 65536.0, 0.0)
        s1, s0 = _sigmoids(dv)
        delta = (-5.0) * s0 * s0 * jnp.log(s1) + s1 * s1 * jnp.log(s0)

        iota_g = jax.lax.broadcasted_iota(jnp.int32, (n_gt, 1), 0)

        def dup_body(g, carry):
            idx_g = bidx[pl.ds(g, 1), 0:1]
            hit = jnp.where(
                jnp.logical_and(bidx_v == idx_g, iota_g < g), validf, 0.0)
            aux[pl.ds(g, 1), :] = jnp.broadcast_to(_sum11(hit), (1, 128))
            return carry

        jax.lax.fori_loop(0, n_gt, dup_body, 0, unroll=False)

        nondup = aux[:, 0:1] < 0.5
        fix = jnp.logical_and(
            jnp.logical_and(validv, jnp.logical_not(tentv)), nondup)
        corr = _sum11(jnp.where(fix, delta, 0.0))

        class_img = sacc[0:1, 0:1] + corr
        sacc[3:4, 0:1] += class_img * (1.0 / jnp.float32(n_total))
        sacc[4:5, 0:1] += coord_img

        @pl.when(i == n_img - 1)
        def _():
            inv_r = 1.0 / jnp.float32(n_img)
            cl = sacc[3:4, 0:1] * inv_r
            co = sacc[4:5, 0:1] * inv_r
            tot = cl + co
            out_t_ref[...] = jnp.broadcast_to(tot, (8, 128))
            out_cl_ref[...] = jnp.broadcast_to(cl, (8, 128))
            out_co_ref[...] = jnp.broadcast_to(co, (8, 128))


def kernel(batch_boxes, batch_classes, anchors, batch_gt, batch_num_objects):
    B, N, _ = batch_boxes.shape
    G = batch_gt.shape[1]
    CH = 4096 if N % 4096 == 0 else N
    K = N // CH

    f32 = jnp.float32
    anch_t = jnp.concatenate(
        [anchors.T.astype(f32), jnp.zeros((4, N), f32)], axis=0)
    bc = jnp.concatenate(
        [jnp.swapaxes(batch_boxes, 1, 2).astype(f32),
         jnp.swapaxes(batch_classes, 1, 2).astype(f32),
         jnp.zeros((B, 2, N), f32)], axis=1)
    valid = (jnp.arange(G)[None, :] < batch_num_objects[:, None]).astype(f32)
    degen = jnp.array([-100.0, -100.0, 1.0, 1.0], f32)
    gt_deg = jnp.where(valid[:, :, None] > 0.5, batch_gt.astype(f32),
                      degen[None, None, :])
    gt_ext = jnp.concatenate(
        [gt_deg, valid[:, :, None], jnp.zeros((B, G, 3), f32)], axis=2)

    body = functools.partial(
        _loss_kernel, n_img=B, n_chunks=K, chunk=CH, n_total=N, n_gt=G)

    out_t, out_cl, out_co = pl.pallas_call(
        body,
        grid=(B, K),
        in_specs=[
            pl.BlockSpec((8, CH), lambda i, k: (0, k)),
            pl.BlockSpec((1, 8, CH), lambda i, k: (i, 0, k)),
            pl.BlockSpec((1, G, 8), lambda i, k: (i, 0, 0)),
        ],
        out_specs=[
            pl.BlockSpec((8, 128), lambda i, k: (0, 0)),
            pl.BlockSpec((8, 128), lambda i, k: (0, 0)),
            pl.BlockSpec((8, 128), lambda i, k: (0, 0)),
        ],
        out_shape=[
            jax.ShapeDtypeStruct((8, 128), f32),
            jax.ShapeDtypeStruct((8, 128), f32),
            jax.ShapeDtypeStruct((8, 128), f32),
        ],
        scratch_shapes=[
            pltpu.VMEM((8, 128), f32),
            pltpu.VMEM((G, 128), f32),
            pltpu.VMEM((G, 128), f32),
            pltpu.VMEM((G, 128), f32),
            pltpu.VMEM((G, 128), f32),
            pltpu.VMEM((G, 128), f32),
        ],
        compiler_params=pltpu.CompilerParams(
            dimension_semantics=("arbitrary", "arbitrary")),
    )(anch_t, bc, gt_ext)

    total = out_t[0, 0]
    cl = out_cl[0, 0]
    co = out_co[0, 0]
    return (total, cl, co)


# speedup vs baseline: 1.0449x; 1.0449x over previous
import functools

import jax
import jax.numpy as jnp
from jax.experimental import pallas as pl
from jax.experimental.pallas import tpu as pltpu

_BIG = 1e9


def _sum11(x):
    return jnp.sum(jnp.sum(x, axis=0, keepdims=True), axis=1, keepdims=True)


def _sigmoids(d):
    ed = jnp.exp(-jnp.abs(d))
    r = 1.0 / (1.0 + ed)
    pos = d >= 0.0
    s1 = jnp.where(pos, r, ed * r)
    s0 = jnp.where(pos, ed * r, r)
    return s1, s0


def _loss_kernel(anch_ref, bc_ref, gt_ref, out_t_ref, out_cl_ref, out_co_ref,
                 sacc, bv, bidx, bl1, bd, aux,
                 *, n_img, n_chunks, chunk, n_total, n_gt):
    i = pl.program_id(0)
    k = pl.program_id(1)

    @pl.when(jnp.logical_and(i == 0, k == 0))
    def _():
        sacc[3:5, :] = jnp.zeros((2, 128), jnp.float32)

    @pl.when(k == 0)
    def _():
        sacc[0:3, :] = jnp.zeros((3, 128), jnp.float32)
        bv[...] = jnp.full((n_gt, 128), -1.0, jnp.float32)

    a_x0 = anch_ref[0:1, :]
    a_y0 = anch_ref[1:2, :]
    a_x1 = anch_ref[2:3, :]
    a_y1 = anch_ref[3:4, :]
    area_a = (a_x1 - a_x0) * (a_y1 - a_y0)

    b0 = bc_ref[0, 0:1, :]
    b1 = bc_ref[0, 1:2, :]
    b2 = bc_ref[0, 2:3, :]
    b3 = bc_ref[0, 3:4, :]
    d = bc_ref[0, 5:6, :] - bc_ref[0, 4:5, :]

    gtb = gt_ref[0]
    g_x0 = gtb[:, 0:1]
    g_y0 = gtb[:, 1:2]
    g_x1 = g_x0 + gtb[:, 2:3]
    g_y1 = g_y0 + gtb[:, 3:4]
    validv = gtb[:, 4:5] > 0.5
    area_g = (g_x1 - g_x0) * (g_y1 - g_y0)

    wx = jnp.maximum(jnp.minimum(a_x1, g_x1) - jnp.maximum(a_x0, g_x0), 0.0)
    wy = jnp.maximum(jnp.minimum(a_y1, g_y1) - jnp.maximum(a_y0, g_y0), 0.0)
    inter = wx * wy
    union = area_a + area_g - inter
    iou = inter / union
    maskf = jnp.where(iou > 0.5, 1.0, 0.0)

    l1 = (jnp.abs(b0 - g_x0) + jnp.abs(b1 - g_y0)
          + jnp.abs(b2 - g_x1) + jnp.abs(b3 - g_y1))

    sacc[1:2, 0:1] += _sum11(maskf * l1)
    cnt_row = jnp.sum(maskf, axis=0, keepdims=True)
    sacc[2:3, 0:1] += jnp.sum(cnt_row, axis=1, keepdims=True)

    anyrow = cnt_row > 0.5
    s1, s0 = _sigmoids(d)
    fl_pos = (-5.0) * s0 * s0 * jnp.log(s1)
    fl_neg = (-1.0) * s1 * s1 * jnp.log(s0)
    fl = jnp.where(anyrow, fl_pos, fl_neg)
    sacc[0:1, 0:1] += _sum11(fl)

    cmax = jnp.max(iou, axis=1, keepdims=True)
    lane = jax.lax.broadcasted_iota(jnp.int32, iou.shape, 1)
    eqm = iou == cmax
    cand_lane = jnp.min(jnp.where(eqm, lane, jnp.int32(2**30)),
                        axis=1, keepdims=True)
    onehot = jnp.where(lane == cand_lane, 1.0, 0.0)
    ext_row = (jnp.clip(d, -1000.0, 1000.0)
               + jnp.where(anyrow, 65536.0, 0.0))
    cand_l1 = jnp.sum(onehot * l1, axis=1, keepdims=True)
    cand_e = jnp.sum(onehot * ext_row, axis=1, keepdims=True)
    cand_idx = (cand_lane + chunk * k).astype(jnp.float32)

    upd = cmax > bv[:, 0:1]
    bv[...] = jnp.where(upd, cmax, bv[...])
    bidx[...] = jnp.where(upd, cand_idx, bidx[...])
    bl1[...] = jnp.where(upd, cand_l1, bl1[...])
    bd[...] = jnp.where(upd, cand_e, bd[...])

    @pl.when(k == n_chunks - 1)
    def _():
        bv_v = bv[:, 0:1]
        bidx_v = bidx[:, 0:1]
        validf = jnp.where(validv, 1.0, 0.0)

        add_pair = jnp.where(jnp.logical_and(validv, bv_v <= 0.5), 1.0, 0.0)
        add_l1 = _sum11(add_pair * bl1[:, 0:1])
        add_cnt = _sum11(add_pair)
        csum_t = sacc[1:2, 0:1] + add_l1
        cnt_t = (sacc[2:3, 0:1] + add_cnt) * 4.0
        coord_img = csum_t / cnt_t

        ev = bd[:, 0:1]
        tentv = ev > 32768.0
        dv = ev - jnp.where(tentv, 65536.0, 0.0)
        s1, s0 = _sigmoids(dv)
        delta = (-5.0) * s0 * s0 * jnp.log(s1) + s1 * s1 * jnp.log(s0)

        iota_g = jax.lax.broadcasted_iota(jnp.int32, (n_gt, 1), 0)

        def dup_body(g, carry):
            idx_g = bidx[pl.ds(g, 1), 0:1]
            hit = jnp.where(
                jnp.logical_and(bidx_v == idx_g, iota_g < g), validf, 0.0)
            aux[pl.ds(g, 1), :] = jnp.broadcast_to(_sum11(hit), (1, 128))
            return carry

        jax.lax.fori_loop(0, n_gt, dup_body, 0, unroll=False)

        nondup = aux[:, 0:1] < 0.5
        fix = jnp.logical_and(
            jnp.logical_and(validv, jnp.logical_not(tentv)), nondup)
        corr = _sum11(jnp.where(fix, delta, 0.0))

        class_img = sacc[0:1, 0:1] + corr
        sacc[3:4, 0:1] += class_img * (1.0 / jnp.float32(n_total))
        sacc[4:5, 0:1] += coord_img

        @pl.when(i == n_img - 1)
        def _():
            inv_r = 1.0 / jnp.float32(n_img)
            cl = sacc[3:4, 0:1] * inv_r
            co = sacc[4:5, 0:1] * inv_r
            tot = cl + co
            out_t_ref[...] = jnp.broadcast_to(tot, (8, 128))
            out_cl_ref[...] = jnp.broadcast_to(cl, (8, 128))
            out_co_ref[...] = jnp.broadcast_to(co, (8, 128))


def kernel(batch_boxes, batch_classes, anchors, batch_gt, batch_num_objects):
    B, N, _ = batch_boxes.shape
    G = batch_gt.shape[1]
    CH = 4096 if N % 4096 == 0 else N
    K = N // CH

    f32 = jnp.float32
    anch_t = jnp.concatenate(
        [anchors.T.astype(f32), jnp.zeros((4, N), f32)], axis=0)
    bc = jnp.concatenate(
        [jnp.swapaxes(batch_boxes, 1, 2).astype(f32),
         jnp.swapaxes(batch_classes, 1, 2).astype(f32),
         jnp.zeros((B, 2, N), f32)], axis=1)
    valid = (jnp.arange(G)[None, :] < batch_num_objects[:, None]).astype(f32)
    degen = jnp.array([-100.0, -100.0, 1.0, 1.0], f32)
    gt_deg = jnp.where(valid[:, :, None] > 0.5, batch_gt.astype(f32),
                      degen[None, None, :])
    gt_ext = jnp.concatenate(
        [gt_deg, valid[:, :, None], jnp.zeros((B, G, 3), f32)], axis=2)

    body = functools.partial(
        _loss_kernel, n_img=B, n_chunks=K, chunk=CH, n_total=N, n_gt=G)

    out_t, out_cl, out_co = pl.pallas_call(
        body,
        grid=(B, K),
        in_specs=[
            pl.BlockSpec((8, CH), lambda i, k: (0, k)),
            pl.BlockSpec((1, 8, CH), lambda i, k: (i, 0, k)),
            pl.BlockSpec((1, G, 8), lambda i, k: (i, 0, 0)),
        ],
        out_specs=[
            pl.BlockSpec((8, 128), lambda i, k: (0, 0)),
            pl.BlockSpec((8, 128), lambda i, k: (0, 0)),
            pl.BlockSpec((8, 128), lambda i, k: (0, 0)),
        ],
        out_shape=[
            jax.ShapeDtypeStruct((8, 128), f32),
            jax.ShapeDtypeStruct((8, 128), f32),
            jax.ShapeDtypeStruct((8, 128), f32),
        ],
        scratch_shapes=[
            pltpu.VMEM((8, 128), f32),
            pltpu.VMEM((G, 128), f32),
            pltpu.VMEM((G, 128), f32),
            pltpu.VMEM((G, 128), f32),
            pltpu.VMEM((G, 128), f32),
            pltpu.VMEM((G, 128), f32),
        ],
        compiler_params=pltpu.CompilerParams(
            dimension_semantics=("arbitrary", "arbitrary")),
    )(anch_t, bc, gt_ext)

    total = out_t[0, 0]
    cl = out_cl[0, 0]
    co = out_co[0, 0]
    return (total, cl, co)


# speedup vs baseline: 1.0804x; 1.0340x over previous
import functools

import jax
import jax.numpy as jnp
from jax.experimental import pallas as pl
from jax.experimental.pallas import tpu as pltpu

_BIG = 1e9


def _sum11(x):
    return jnp.sum(jnp.sum(x, axis=0, keepdims=True), axis=1, keepdims=True)


def _sigmoids(d):
    ed = jnp.exp(-jnp.abs(d))
    r = 1.0 / (1.0 + ed)
    pos = d >= 0.0
    s1 = jnp.where(pos, r, ed * r)
    s0 = jnp.where(pos, ed * r, r)
    return s1, s0


def _loss_kernel(anch_ref, bx_ref, cl_ref, gt_ref,
                 out_t_ref, out_cl_ref, out_co_ref,
                 sacc, bv, bidx, bl1, bd, aux,
                 *, n_img, n_chunks, chunk, n_total, n_gt):
    i = pl.program_id(0)
    k = pl.program_id(1)

    @pl.when(jnp.logical_and(i == 0, k == 0))
    def _():
        sacc[3:5, :] = jnp.zeros((2, 128), jnp.float32)

    @pl.when(k == 0)
    def _():
        sacc[0:3, :] = jnp.zeros((3, 128), jnp.float32)
        bv[...] = jnp.full((n_gt, 128), -1.0, jnp.float32)

    a_x0 = anch_ref[0:1, :]
    a_y0 = anch_ref[1:2, :]
    a_x1 = anch_ref[2:3, :]
    a_y1 = anch_ref[3:4, :]
    area_a = (a_x1 - a_x0) * (a_y1 - a_y0)

    b0 = bx_ref[0, 0:1, :]
    b1 = bx_ref[0, 1:2, :]
    b2 = bx_ref[0, 2:3, :]
    b3 = bx_ref[0, 3:4, :]
    d = cl_ref[0, 1:2, :] - cl_ref[0, 0:1, :]

    gtb = gt_ref[0]
    g_x0 = gtb[:, 0:1]
    g_y0 = gtb[:, 1:2]
    g_x1 = g_x0 + gtb[:, 2:3]
    g_y1 = g_y0 + gtb[:, 3:4]
    validv = gtb[:, 4:5] > 0.5
    area_g = (g_x1 - g_x0) * (g_y1 - g_y0)

    wx = jnp.maximum(jnp.minimum(a_x1, g_x1) - jnp.maximum(a_x0, g_x0), 0.0)
    wy = jnp.maximum(jnp.minimum(a_y1, g_y1) - jnp.maximum(a_y0, g_y0), 0.0)
    inter = wx * wy
    union = area_a + area_g - inter
    iou = inter / union
    maskf = jnp.where(iou > 0.5, 1.0, 0.0)

    l1 = (jnp.abs(b0 - g_x0) + jnp.abs(b1 - g_y0)
          + jnp.abs(b2 - g_x1) + jnp.abs(b3 - g_y1))

    sacc[1:2, 0:1] += _sum11(maskf * l1)
    cnt_row = jnp.sum(maskf, axis=0, keepdims=True)
    sacc[2:3, 0:1] += jnp.sum(cnt_row, axis=1, keepdims=True)

    anyrow = cnt_row > 0.5
    s1, s0 = _sigmoids(d)
    fl_pos = (-5.0) * s0 * s0 * jnp.log(s1)
    fl_neg = (-1.0) * s1 * s1 * jnp.log(s0)
    fl = jnp.where(anyrow, fl_pos, fl_neg)
    sacc[0:1, 0:1] += _sum11(fl)

    cmax = jnp.max(iou, axis=1, keepdims=True)
    lane = jax.lax.broadcasted_iota(jnp.int32, iou.shape, 1)
    eqm = iou == cmax
    cand_lane = jnp.min(jnp.where(eqm, lane, jnp.int32(2**30)),
                        axis=1, keepdims=True)
    onehot = jnp.where(lane == cand_lane, 1.0, 0.0)
    ext_row = (jnp.clip(d, -1000.0, 1000.0)
               + jnp.where(anyrow, 65536.0, 0.0))
    cand_l1 = jnp.sum(onehot * l1, axis=1, keepdims=True)
    cand_e = jnp.sum(onehot * ext_row, axis=1, keepdims=True)
    cand_idx = (cand_lane + chunk * k).astype(jnp.float32)

    upd = cmax > bv[:, 0:1]
    bv[...] = jnp.where(upd, cmax, bv[...])
    bidx[...] = jnp.where(upd, cand_idx, bidx[...])
    bl1[...] = jnp.where(upd, cand_l1, bl1[...])
    bd[...] = jnp.where(upd, cand_e, bd[...])

    @pl.when(k == n_chunks - 1)
    def _():
        bv_v = bv[:, 0:1]
        bidx_v = bidx[:, 0:1]
        validf = jnp.where(validv, 1.0, 0.0)

        add_pair = jnp.where(jnp.logical_and(validv, bv_v <= 0.5), 1.0, 0.0)
        add_l1 = _sum11(add_pair * bl1[:, 0:1])
        add_cnt = _sum11(add_pair)
        csum_t = sacc[1:2, 0:1] + add_l1
        cnt_t = (sacc[2:3, 0:1] + add_cnt) * 4.0
        coord_img = csum_t / cnt_t

        ev = bd[:, 0:1]
        tentv = ev > 32768.0
        dv = ev - jnp.where(tentv, 65536.0, 0.0)
        s1, s0 = _sigmoids(dv)
        delta = (-5.0) * s0 * s0 * jnp.log(s1) + s1 * s1 * jnp.log(s0)

        iota_g = jax.lax.broadcasted_iota(jnp.int32, (n_gt, 1), 0)

        def dup_body(g, carry):
            idx_g = bidx[pl.ds(g, 1), 0:1]
            hit = jnp.where(
                jnp.logical_and(bidx_v == idx_g, iota_g < g), validf, 0.0)
            aux[pl.ds(g, 1), :] = jnp.broadcast_to(_sum11(hit), (1, 128))
            return carry

        jax.lax.fori_loop(0, n_gt, dup_body, 0, unroll=False)

        nondup = aux[:, 0:1] < 0.5
        fix = jnp.logical_and(
            jnp.logical_and(validv, jnp.logical_not(tentv)), nondup)
        corr = _sum11(jnp.where(fix, delta, 0.0))

        class_img = sacc[0:1, 0:1] + corr
        sacc[3:4, 0:1] += class_img * (1.0 / jnp.float32(n_total))
        sacc[4:5, 0:1] += coord_img

        @pl.when(i == n_img - 1)
        def _():
            inv_r = 1.0 / jnp.float32(n_img)
            cl = sacc[3:4, 0:1] * inv_r
            co = sacc[4:5, 0:1] * inv_r
            tot = cl + co
            out_t_ref[...] = jnp.broadcast_to(tot, (8, 128))
            out_cl_ref[...] = jnp.broadcast_to(cl, (8, 128))
            out_co_ref[...] = jnp.broadcast_to(co, (8, 128))


def kernel(batch_boxes, batch_classes, anchors, batch_gt, batch_num_objects):
    B, N, _ = batch_boxes.shape
    G = batch_gt.shape[1]
    CH = 4096 if N % 4096 == 0 else N
    K = N // CH

    f32 = jnp.float32
    anch_t = anchors.T.astype(f32)
    bx_t = jnp.swapaxes(batch_boxes, 1, 2).astype(f32)
    cl_t = jnp.swapaxes(batch_classes, 1, 2).astype(f32)
    valid = (jnp.arange(G)[None, :] < batch_num_objects[:, None]).astype(f32)
    degen = jnp.array([-100.0, -100.0, 1.0, 1.0], f32)
    gt_deg = jnp.where(valid[:, :, None] > 0.5, batch_gt.astype(f32),
                      degen[None, None, :])
    gt_ext = jnp.concatenate(
        [gt_deg, valid[:, :, None], jnp.zeros((B, G, 3), f32)], axis=2)

    body = functools.partial(
        _loss_kernel, n_img=B, n_chunks=K, chunk=CH, n_total=N, n_gt=G)

    out_t, out_cl, out_co = pl.pallas_call(
        body,
        grid=(B, K),
        in_specs=[
            pl.BlockSpec((4, CH), lambda i, k: (0, k)),
            pl.BlockSpec((1, 4, CH), lambda i, k: (i, 0, k)),
            pl.BlockSpec((1, 2, CH), lambda i, k: (i, 0, k)),
            pl.BlockSpec((1, G, 8), lambda i, k: (i, 0, 0)),
        ],
        out_specs=[
            pl.BlockSpec((8, 128), lambda i, k: (0, 0)),
            pl.BlockSpec((8, 128), lambda i, k: (0, 0)),
            pl.BlockSpec((8, 128), lambda i, k: (0, 0)),
        ],
        out_shape=[
            jax.ShapeDtypeStruct((8, 128), f32),
            jax.ShapeDtypeStruct((8, 128), f32),
            jax.ShapeDtypeStruct((8, 128), f32),
        ],
        scratch_shapes=[
            pltpu.VMEM((8, 128), f32),
            pltpu.VMEM((G, 128), f32),
            pltpu.VMEM((G, 128), f32),
            pltpu.VMEM((G, 128), f32),
            pltpu.VMEM((G, 128), f32),
            pltpu.VMEM((G, 128), f32),
        ],
        compiler_params=pltpu.CompilerParams(
            dimension_semantics=("arbitrary", "arbitrary")),
    )(anch_t, bx_t, cl_t, gt_ext)

    total = out_t[0, 0]
    cl = out_cl[0, 0]
    co = out_co[0, 0]
    return (total, cl, co)


# speedup vs baseline: 1.3835x; 1.2806x over previous
import functools

import jax
import jax.numpy as jnp
from jax.experimental import pallas as pl
from jax.experimental.pallas import tpu as pltpu

_BIG = 1e9


def _sum11(x):
    return jnp.sum(jnp.sum(x, axis=0, keepdims=True), axis=1, keepdims=True)


def _sigmoids(d):
    ed = jnp.exp(-jnp.abs(d))
    r = 1.0 / (1.0 + ed)
    pos = d >= 0.0
    s1 = jnp.where(pos, r, ed * r)
    s0 = jnp.where(pos, ed * r, r)
    return s1, s0


def _loss_kernel(anch_ref, bx_ref, cl_ref, gt_ref,
                 out_t_ref, out_cl_ref, out_co_ref,
                 sacc, bv, bidx, bl1, bd,
                 *, n_img, n_chunks, chunk, n_total, n_gt):
    i = pl.program_id(0)
    k = pl.program_id(1)

    @pl.when(jnp.logical_and(i == 0, k == 0))
    def _():
        sacc[3:5, :] = jnp.zeros((2, 128), jnp.float32)

    @pl.when(k == 0)
    def _():
        sacc[0:3, :] = jnp.zeros((3, 128), jnp.float32)
        bv[...] = jnp.full((n_gt, 128), -1.0, jnp.float32)

    a_x0 = anch_ref[0:1, :]
    a_y0 = anch_ref[1:2, :]
    a_x1 = anch_ref[2:3, :]
    a_y1 = anch_ref[3:4, :]
    area_a = (a_x1 - a_x0) * (a_y1 - a_y0)

    b0 = bx_ref[0, 0:1, :]
    b1 = bx_ref[0, 1:2, :]
    b2 = bx_ref[0, 2:3, :]
    b3 = bx_ref[0, 3:4, :]
    d = cl_ref[0, 1:2, :] - cl_ref[0, 0:1, :]

    gtb = gt_ref[0]
    g_x0 = gtb[:, 0:1]
    g_y0 = gtb[:, 1:2]
    g_x1 = g_x0 + gtb[:, 2:3]
    g_y1 = g_y0 + gtb[:, 3:4]
    validv = gtb[:, 4:5] > 0.5
    area_g = (g_x1 - g_x0) * (g_y1 - g_y0)

    wx = jnp.maximum(jnp.minimum(a_x1, g_x1) - jnp.maximum(a_x0, g_x0), 0.0)
    wy = jnp.maximum(jnp.minimum(a_y1, g_y1) - jnp.maximum(a_y0, g_y0), 0.0)
    inter = wx * wy
    union = area_a + area_g - inter
    iou = inter / union
    maskf = jnp.where(iou > 0.5, 1.0, 0.0)

    l1 = (jnp.abs(b0 - g_x0) + jnp.abs(b1 - g_y0)
          + jnp.abs(b2 - g_x1) + jnp.abs(b3 - g_y1))

    sacc[1:2, 0:1] += _sum11(maskf * l1)
    cnt_row = jnp.sum(maskf, axis=0, keepdims=True)
    sacc[2:3, 0:1] += jnp.sum(cnt_row, axis=1, keepdims=True)

    anyrow = cnt_row > 0.5
    s1, s0 = _sigmoids(d)
    fl_pos = (-5.0) * s0 * s0 * jnp.log(s1)
    fl_neg = (-1.0) * s1 * s1 * jnp.log(s0)
    fl = jnp.where(anyrow, fl_pos, fl_neg)
    sacc[0:1, 0:1] += _sum11(fl)

    cmax = jnp.max(iou, axis=1, keepdims=True)
    lane = jax.lax.broadcasted_iota(jnp.int32, iou.shape, 1)
    eqm = iou == cmax
    cand_lane = jnp.min(jnp.where(eqm, lane, jnp.int32(2**30)),
                        axis=1, keepdims=True)
    onehot = jnp.where(lane == cand_lane, 1.0, 0.0)
    ext_row = (jnp.clip(d, -1000.0, 1000.0)
               + jnp.where(anyrow, 65536.0, 0.0))
    cand_l1 = jnp.sum(onehot * l1, axis=1, keepdims=True)
    cand_e = jnp.sum(onehot * ext_row, axis=1, keepdims=True)
    cand_idx = (cand_lane + chunk * k).astype(jnp.float32)

    upd = cmax > bv[:, 0:1]
    bv[...] = jnp.where(upd, cmax, bv[...])
    bidx[...] = jnp.where(upd, cand_idx, bidx[...])
    bl1[...] = jnp.where(upd, cand_l1, bl1[...])
    bd[...] = jnp.where(upd, cand_e, bd[...])

    @pl.when(k == n_chunks - 1)
    def _():
        bv_v = bv[:, 0:1]
        bidx_v = bidx[:, 0:1]
        validf = jnp.where(validv, 1.0, 0.0)

        add_pair = jnp.where(jnp.logical_and(validv, bv_v <= 0.5), 1.0, 0.0)
        add_l1 = _sum11(add_pair * bl1[:, 0:1])
        add_cnt = _sum11(add_pair)
        csum_t = sacc[1:2, 0:1] + add_l1
        cnt_t = (sacc[2:3, 0:1] + add_cnt) * 4.0
        coord_img = csum_t / cnt_t

        ev = bd[:, 0:1]
        tentv = ev > 32768.0
        dv = ev - jnp.where(tentv, 65536.0, 0.0)
        s1, s0 = _sigmoids(dv)
        delta = (-5.0) * s0 * s0 * jnp.log(s1) + s1 * s1 * jnp.log(s0)

        iota_r = jax.lax.broadcasted_iota(jnp.int32, (n_gt, n_gt), 0)
        iota_c = jax.lax.broadcasted_iota(jnp.int32, (n_gt, n_gt), 1)
        diag = jnp.where(iota_r == iota_c, 1.0, 0.0)
        bidx_row = jnp.sum(diag * bidx_v, axis=0, keepdims=True)
        valid_row = jnp.sum(diag * validf, axis=0, keepdims=True)
        dupmat = jnp.where(
            jnp.logical_and(
                jnp.logical_and(bidx_v == bidx_row, valid_row > 0.5),
                iota_c < iota_r),
            1.0, 0.0)
        nondup = jnp.sum(dupmat, axis=1, keepdims=True) < 0.5
        fix = jnp.logical_and(
            jnp.logical_and(validv, jnp.logical_not(tentv)), nondup)
        corr = _sum11(jnp.where(fix, delta, 0.0))

        class_img = sacc[0:1, 0:1] + corr
        sacc[3:4, 0:1] += class_img * (1.0 / jnp.float32(n_total))
        sacc[4:5, 0:1] += coord_img

        @pl.when(i == n_img - 1)
        def _():
            inv_r = 1.0 / jnp.float32(n_img)
            cl = sacc[3:4, 0:1] * inv_r
            co = sacc[4:5, 0:1] * inv_r
            tot = cl + co
            out_t_ref[...] = jnp.broadcast_to(tot, (8, 128))
            out_cl_ref[...] = jnp.broadcast_to(cl, (8, 128))
            out_co_ref[...] = jnp.broadcast_to(co, (8, 128))


def kernel(batch_boxes, batch_classes, anchors, batch_gt, batch_num_objects):
    B, N, _ = batch_boxes.shape
    G = batch_gt.shape[1]
    CH = 4096 if N % 4096 == 0 else N
    K = N // CH

    f32 = jnp.float32
    anch_t = anchors.T.astype(f32)
    bx_t = jnp.swapaxes(batch_boxes, 1, 2).astype(f32)
    cl_t = jnp.swapaxes(batch_classes, 1, 2).astype(f32)
    valid = (jnp.arange(G)[None, :] < batch_num_objects[:, None]).astype(f32)
    degen = jnp.array([-100.0, -100.0, 1.0, 1.0], f32)
    gt_deg = jnp.where(valid[:, :, None] > 0.5, batch_gt.astype(f32),
                      degen[None, None, :])
    gt_ext = jnp.concatenate(
        [gt_deg, valid[:, :, None], jnp.zeros((B, G, 3), f32)], axis=2)

    body = functools.partial(
        _loss_kernel, n_img=B, n_chunks=K, chunk=CH, n_total=N, n_gt=G)

    out_t, out_cl, out_co = pl.pallas_call(
        body,
        grid=(B, K),
        in_specs=[
            pl.BlockSpec((4, CH), lambda i, k: (0, k)),
            pl.BlockSpec((1, 4, CH), lambda i, k: (i, 0, k)),
            pl.BlockSpec((1, 2, CH), lambda i, k: (i, 0, k)),
            pl.BlockSpec((1, G, 8), lambda i, k: (i, 0, 0)),
        ],
        out_specs=[
            pl.BlockSpec((8, 128), lambda i, k: (0, 0)),
            pl.BlockSpec((8, 128), lambda i, k: (0, 0)),
            pl.BlockSpec((8, 128), lambda i, k: (0, 0)),
        ],
        out_shape=[
            jax.ShapeDtypeStruct((8, 128), f32),
            jax.ShapeDtypeStruct((8, 128), f32),
            jax.ShapeDtypeStruct((8, 128), f32),
        ],
        scratch_shapes=[
            pltpu.VMEM((8, 128), f32),
            pltpu.VMEM((G, 128), f32),
            pltpu.VMEM((G, 128), f32),
            pltpu.VMEM((G, 128), f32),
            pltpu.VMEM((G, 128), f32),
        ],
        compiler_params=pltpu.CompilerParams(
            dimension_semantics=("arbitrary", "arbitrary")),
    )(anch_t, bx_t, cl_t, gt_ext)

    total = out_t[0, 0]
    cl = out_cl[0, 0]
    co = out_co[0, 0]
    return (total, cl, co)


# speedup vs baseline: 1.5104x; 1.0917x over previous
import functools

import jax
import jax.numpy as jnp
from jax.experimental import pallas as pl
from jax.experimental.pallas import tpu as pltpu

_BIG = 1e9


def _sum11(x):
    return jnp.sum(jnp.sum(x, axis=0, keepdims=True), axis=1, keepdims=True)


def _lanefold(x):
    parts = [x[:, j * 128:(j + 1) * 128] for j in range(x.shape[1] // 128)]
    while len(parts) > 1:
        h = len(parts) // 2
        merged = [parts[j] + parts[h + j] for j in range(h)]
        if len(parts) % 2:
            merged.append(parts[-1])
        parts = merged
    return parts[0]


def _sigmoids(d):
    ed = jnp.exp(-jnp.abs(d))
    r = 1.0 / (1.0 + ed)
    pos = d >= 0.0
    s1 = jnp.where(pos, r, ed * r)
    s0 = jnp.where(pos, ed * r, r)
    return s1, s0


def _loss_kernel(anch_ref, bx_ref, cl_ref, gt_ref,
                 out_t_ref, out_cl_ref, out_co_ref,
                 sacc, macc, bv, bidx, bl1, bd,
                 *, n_img, n_chunks, chunk, n_total, n_gt):
    i = pl.program_id(0)
    k = pl.program_id(1)

    @pl.when(jnp.logical_and(i == 0, k == 0))
    def _():
        sacc[3:5, :] = jnp.zeros((2, 128), jnp.float32)

    @pl.when(k == 0)
    def _():
        sacc[0:3, :] = jnp.zeros((3, 128), jnp.float32)
        macc[...] = jnp.zeros((n_gt, 128), jnp.float32)
        bv[...] = jnp.full((n_gt, 128), -1.0, jnp.float32)

    a_x0 = anch_ref[0:1, :]
    a_y0 = anch_ref[1:2, :]
    a_x1 = anch_ref[2:3, :]
    a_y1 = anch_ref[3:4, :]
    area_a = (a_x1 - a_x0) * (a_y1 - a_y0)

    b0 = bx_ref[0, 0:1, :]
    b1 = bx_ref[0, 1:2, :]
    b2 = bx_ref[0, 2:3, :]
    b3 = bx_ref[0, 3:4, :]
    d = cl_ref[0, 1:2, :] - cl_ref[0, 0:1, :]

    gtb = gt_ref[0]
    g_x0 = gtb[:, 0:1]
    g_y0 = gtb[:, 1:2]
    g_x1 = g_x0 + gtb[:, 2:3]
    g_y1 = g_y0 + gtb[:, 3:4]
    validv = gtb[:, 4:5] > 0.5
    area_g = (g_x1 - g_x0) * (g_y1 - g_y0)

    wx = jnp.maximum(jnp.minimum(a_x1, g_x1) - jnp.maximum(a_x0, g_x0), 0.0)
    wy = jnp.maximum(jnp.minimum(a_y1, g_y1) - jnp.maximum(a_y0, g_y0), 0.0)
    inter = wx * wy
    union = area_a + area_g - inter
    iou = inter / union
    maskf = jnp.where(iou > 0.5, 1.0, 0.0)

    bf = jnp.bfloat16
    l1 = ((jnp.abs(b0.astype(bf) - g_x0.astype(bf))
           + jnp.abs(b1.astype(bf) - g_y0.astype(bf)))
          + (jnp.abs(b2.astype(bf) - g_x1.astype(bf))
             + jnp.abs(b3.astype(bf) - g_y1.astype(bf)))).astype(jnp.float32)

    macc[...] += _lanefold(maskf * l1)
    cnt_row = jnp.sum(maskf, axis=0, keepdims=True)
    sacc[2:3, :] += _lanefold(cnt_row)

    anyrow = cnt_row > 0.5
    s1, s0 = _sigmoids(d)
    fl_pos = (-5.0) * s0 * s0 * jnp.log(s1)
    fl_neg = (-1.0) * s1 * s1 * jnp.log(s0)
    fl = jnp.where(anyrow, fl_pos, fl_neg)
    sacc[0:1, :] += _lanefold(fl)

    cmax = jnp.max(iou, axis=1, keepdims=True)
    lane = jax.lax.broadcasted_iota(jnp.int32, iou.shape, 1)
    eqm = iou == cmax
    cand_lane = jnp.min(jnp.where(eqm, lane, jnp.int32(2**30)),
                        axis=1, keepdims=True)
    onehot = jnp.where(lane == cand_lane, 1.0, 0.0)
    ext_row = (jnp.clip(d, -1000.0, 1000.0)
               + jnp.where(anyrow, 65536.0, 0.0))
    cand_l1 = jnp.sum(onehot * l1, axis=1, keepdims=True)
    cand_e = jnp.sum(onehot * ext_row, axis=1, keepdims=True)
    cand_idx = (cand_lane + chunk * k).astype(jnp.float32)

    upd = cmax > bv[:, 0:1]
    bv[...] = jnp.where(upd, cmax, bv[...])
    bidx[...] = jnp.where(upd, cand_idx, bidx[...])
    bl1[...] = jnp.where(upd, cand_l1, bl1[...])
    bd[...] = jnp.where(upd, cand_e, bd[...])

    @pl.when(k == n_chunks - 1)
    def _():
        bv_v = bv[:, 0:1]
        bidx_v = bidx[:, 0:1]
        validf = jnp.where(validv, 1.0, 0.0)

        add_pair = jnp.where(jnp.logical_and(validv, bv_v <= 0.5), 1.0, 0.0)
        add_l1 = _sum11(add_pair * bl1[:, 0:1])
        add_cnt = _sum11(add_pair)
        csum_t = _sum11(macc[...]) + add_l1
        cnt_t = (jnp.sum(sacc[2:3, :], axis=1, keepdims=True) + add_cnt) * 4.0
        coord_img = csum_t / cnt_t

        ev = bd[:, 0:1]
        tentv = ev > 32768.0
        dv = ev - jnp.where(tentv, 65536.0, 0.0)
        s1, s0 = _sigmoids(dv)
        delta = (-5.0) * s0 * s0 * jnp.log(s1) + s1 * s1 * jnp.log(s0)

        iota_r = jax.lax.broadcasted_iota(jnp.int32, (n_gt, n_gt), 0)
        iota_c = jax.lax.broadcasted_iota(jnp.int32, (n_gt, n_gt), 1)
        diag = jnp.where(iota_r == iota_c, 1.0, 0.0)
        bidx_row = jnp.sum(diag * bidx_v, axis=0, keepdims=True)
        valid_row = jnp.sum(diag * validf, axis=0, keepdims=True)
        dupmat = jnp.where(
            jnp.logical_and(
                jnp.logical_and(bidx_v == bidx_row, valid_row > 0.5),
                iota_c < iota_r),
            1.0, 0.0)
        nondup = jnp.sum(dupmat, axis=1, keepdims=True) < 0.5
        fix = jnp.logical_and(
            jnp.logical_and(validv, jnp.logical_not(tentv)), nondup)
        corr = _sum11(jnp.where(fix, delta, 0.0))

        class_img = jnp.sum(sacc[0:1, :], axis=1, keepdims=True) + corr
        sacc[3:4, 0:1] += class_img * (1.0 / jnp.float32(n_total))
        sacc[4:5, 0:1] += coord_img

        @pl.when(i == n_img - 1)
        def _():
            inv_r = 1.0 / jnp.float32(n_img)
            cl = sacc[3:4, 0:1] * inv_r
            co = sacc[4:5, 0:1] * inv_r
            tot = cl + co
            out_t_ref[...] = jnp.broadcast_to(tot, (8, 128))
            out_cl_ref[...] = jnp.broadcast_to(cl, (8, 128))
            out_co_ref[...] = jnp.broadcast_to(co, (8, 128))


def kernel(batch_boxes, batch_classes, anchors, batch_gt, batch_num_objects):
    B, N, _ = batch_boxes.shape
    G = batch_gt.shape[1]
    CH = 4096 if N % 4096 == 0 else N
    K = N // CH

    f32 = jnp.float32
    anch_t = anchors.T.astype(f32)
    bx_t = jnp.swapaxes(batch_boxes, 1, 2).astype(f32)
    cl_t = jnp.swapaxes(batch_classes, 1, 2).astype(f32)
    valid = (jnp.arange(G)[None, :] < batch_num_objects[:, None]).astype(f32)
    degen = jnp.array([-100.0, -100.0, 1.0, 1.0], f32)
    gt_deg = jnp.where(valid[:, :, None] > 0.5, batch_gt.astype(f32),
                      degen[None, None, :])
    gt_ext = jnp.concatenate(
        [gt_deg, valid[:, :, None], jnp.zeros((B, G, 3), f32)], axis=2)

    body = functools.partial(
        _loss_kernel, n_img=B, n_chunks=K, chunk=CH, n_total=N, n_gt=G)

    out_t, out_cl, out_co = pl.pallas_call(
        body,
        grid=(B, K),
        in_specs=[
            pl.BlockSpec((4, CH), lambda i, k: (0, k)),
            pl.BlockSpec((1, 4, CH), lambda i, k: (i, 0, k)),
            pl.BlockSpec((1, 2, CH), lambda i, k: (i, 0, k)),
            pl.BlockSpec((1, G, 8), lambda i, k: (i, 0, 0)),
        ],
        out_specs=[
            pl.BlockSpec((8, 128), lambda i, k: (0, 0)),
            pl.BlockSpec((8, 128), lambda i, k: (0, 0)),
            pl.BlockSpec((8, 128), lambda i, k: (0, 0)),
        ],
        out_shape=[
            jax.ShapeDtypeStruct((8, 128), f32),
            jax.ShapeDtypeStruct((8, 128), f32),
            jax.ShapeDtypeStruct((8, 128), f32),
        ],
        scratch_shapes=[
            pltpu.VMEM((8, 128), f32),
            pltpu.VMEM((G, 128), f32),
            pltpu.VMEM((G, 128), f32),
            pltpu.VMEM((G, 128), f32),
            pltpu.VMEM((G, 128), f32),
            pltpu.VMEM((G, 128), f32),
        ],
        compiler_params=pltpu.CompilerParams(
            dimension_semantics=("arbitrary", "arbitrary")),
    )(anch_t, bx_t, cl_t, gt_ext)

    total = out_t[0, 0]
    cl = out_cl[0, 0]
    co = out_co[0, 0]
    return (total, cl, co)


# speedup vs baseline: 1.6482x; 1.0912x over previous
import functools

import jax
import jax.numpy as jnp
from jax.experimental import pallas as pl
from jax.experimental.pallas import tpu as pltpu


def _sum11(x):
    return jnp.sum(jnp.sum(x, axis=0, keepdims=True), axis=1, keepdims=True)


def _lanefold(x):
    parts = [x[:, j * 128:(j + 1) * 128] for j in range(x.shape[1] // 128)]
    while len(parts) > 1:
        h = len(parts) // 2
        merged = [parts[j] + parts[h + j] for j in range(h)]
        if len(parts) % 2:
            merged.append(parts[-1])
        parts = merged
    return parts[0]


def _sigmoids(d):
    ed = jnp.exp(-jnp.abs(d))
    r = 1.0 / (1.0 + ed)
    pos = d >= 0.0
    s1 = jnp.where(pos, r, ed * r)
    s0 = jnp.where(pos, ed * r, r)
    return s1, s0


def _loss_kernel(anch_ref, bx_ref, cl_ref, gt_ref,
                 out_t_ref, out_cl_ref, out_co_ref,
                 sacc, macc, bv, bidx, bl1, bd,
                 *, n_steps, n_chunks, n_im, chunk, n_total, n_gt):
    i = pl.program_id(0)
    k = pl.program_id(1)

    @pl.when(jnp.logical_and(i == 0, k == 0))
    def _():
        sacc[2 * n_im:2 * n_im + 2, :] = jnp.zeros((2, 128), jnp.float32)

    @pl.when(k == 0)
    def _():
        sacc[0:2 * n_im, :] = jnp.zeros((2 * n_im, 128), jnp.float32)
        macc[...] = jnp.zeros((n_im * n_gt, 128), jnp.float32)
        bv[...] = jnp.full((n_im * n_gt, 128), -1.0, jnp.float32)

    a_x0 = anch_ref[0:1, :]
    a_y0 = anch_ref[1:2, :]
    a_x1 = anch_ref[2:3, :]
    a_y1 = anch_ref[3:4, :]
    area_a = (a_x1 - a_x0) * (a_y1 - a_y0)
    lane = jax.lax.broadcasted_iota(jnp.int32, (n_gt, chunk), 1)
    bf = jnp.bfloat16

    for j in range(n_im):
        r0, r1 = j * n_gt, (j + 1) * n_gt

        b0 = bx_ref[j, 0:1, :]
        b1 = bx_ref[j, 1:2, :]
        b2 = bx_ref[j, 2:3, :]
        b3 = bx_ref[j, 3:4, :]
        d = cl_ref[j, 1:2, :] - cl_ref[j, 0:1, :]

        gtb = gt_ref[j]
        g_x0 = gtb[:, 0:1]
        g_y0 = gtb[:, 1:2]
        g_x1 = g_x0 + gtb[:, 2:3]
        g_y1 = g_y0 + gtb[:, 3:4]
        area_g = (g_x1 - g_x0) * (g_y1 - g_y0)

        wx = jnp.maximum(
            jnp.minimum(a_x1, g_x1) - jnp.maximum(a_x0, g_x0), 0.0)
        wy = jnp.maximum(
            jnp.minimum(a_y1, g_y1) - jnp.maximum(a_y0, g_y0), 0.0)
        inter = wx * wy
        union = area_a + area_g - inter
        iou = inter / union
        maskf = jnp.where(iou > 0.5, 1.0, 0.0)

        l1 = ((jnp.abs(b0.astype(bf) - g_x0.astype(bf))
               + jnp.abs(b1.astype(bf) - g_y0.astype(bf)))
              + (jnp.abs(b2.astype(bf) - g_x1.astype(bf))
                 + jnp.abs(b3.astype(bf) - g_y1.astype(bf)))
              ).astype(jnp.float32)

        macc[r0:r1, :] += _lanefold(maskf * l1)
        cnt_row = jnp.sum(maskf, axis=0, keepdims=True)
        sacc[n_im + j:n_im + j + 1, :] += _lanefold(cnt_row)

        anyrow = cnt_row > 0.5
        s1, s0 = _sigmoids(d)
        fl_pos = (-5.0) * s0 * s0 * jnp.log(s1)
        fl_neg = (-1.0) * s1 * s1 * jnp.log(s0)
        fl = jnp.where(anyrow, fl_pos, fl_neg)
        sacc[j:j + 1, :] += _lanefold(fl)

        cmax = jnp.max(iou, axis=1, keepdims=True)
        eqm = iou == cmax
        cand_lane = jnp.min(jnp.where(eqm, lane, jnp.int32(2**30)),
                            axis=1, keepdims=True)
        onehot = jnp.where(lane == cand_lane, 1.0, 0.0)
        ext_row = (jnp.clip(d, -1000.0, 1000.0)
                   + jnp.where(anyrow, 65536.0, 0.0))
        cand_l1 = jnp.sum(onehot * l1, axis=1, keepdims=True)
        cand_e = jnp.sum(onehot * ext_row, axis=1, keepdims=True)
        cand_idx = (cand_lane + chunk * k).astype(jnp.float32)

        upd = cmax > bv[r0:r1, 0:1]
        bv[r0:r1, :] = jnp.where(upd, cmax, bv[r0:r1, :])
        bidx[r0:r1, :] = jnp.where(upd, cand_idx, bidx[r0:r1, :])
        bl1[r0:r1, :] = jnp.where(upd, cand_l1, bl1[r0:r1, :])
        bd[r0:r1, :] = jnp.where(upd, cand_e, bd[r0:r1, :])

    @pl.when(k == n_chunks - 1)
    def _():
        for j in range(n_im):
            r0, r1 = j * n_gt, (j + 1) * n_gt
            validv = gt_ref[j][:, 4:5] > 0.5
            bv_v = bv[r0:r1, 0:1]
            bidx_v = bidx[r0:r1, 0:1]
            validf = jnp.where(validv, 1.0, 0.0)

            add_pair = jnp.where(
                jnp.logical_and(validv, bv_v <= 0.5), 1.0, 0.0)
            add_l1 = _sum11(add_pair * bl1[r0:r1, 0:1])
            add_cnt = _sum11(add_pair)
            csum_t = _sum11(macc[r0:r1, :]) + add_l1
            cnt_t = (jnp.sum(sacc[n_im + j:n_im + j + 1, :],
                             axis=1, keepdims=True) + add_cnt) * 4.0
            coord_img = csum_t / cnt_t

            ev = bd[r0:r1, 0:1]
            tentv = ev > 32768.0
            dv = ev - jnp.where(tentv, 65536.0, 0.0)
            s1, s0 = _sigmoids(dv)
            delta = ((-5.0) * s0 * s0 * jnp.log(s1)
                     + s1 * s1 * jnp.log(s0))

            iota_r = jax.lax.broadcasted_iota(jnp.int32, (n_gt, n_gt), 0)
            iota_c = jax.lax.broadcasted_iota(jnp.int32, (n_gt, n_gt), 1)
            diag = jnp.where(iota_r == iota_c, 1.0, 0.0)
            bidx_row = jnp.sum(diag * bidx_v, axis=0, keepdims=True)
            valid_row = jnp.sum(diag * validf, axis=0, keepdims=True)
            dupmat = jnp.where(
                jnp.logical_and(
                    jnp.logical_and(bidx_v == bidx_row, valid_row > 0.5),
                    iota_c < iota_r),
                1.0, 0.0)
            nondup = jnp.sum(dupmat, axis=1, keepdims=True) < 0.5
            fix = jnp.logical_and(
                jnp.logical_and(validv, jnp.logical_not(tentv)), nondup)
            corr = _sum11(jnp.where(fix, delta, 0.0))

            class_img = (jnp.sum(sacc[j:j + 1, :], axis=1, keepdims=True)
                         + corr)
            sacc[2 * n_im:2 * n_im + 1, 0:1] += (
                class_img * (1.0 / jnp.float32(n_total)))
            sacc[2 * n_im + 1:2 * n_im + 2, 0:1] += coord_img

        @pl.when(i == n_steps - 1)
        def _():
            inv_r = 1.0 / jnp.float32(n_steps * n_im)
            cl = sacc[2 * n_im:2 * n_im + 1, 0:1] * inv_r
            co = sacc[2 * n_im + 1:2 * n_im + 2, 0:1] * inv_r
            tot = cl + co
            out_t_ref[...] = jnp.broadcast_to(tot, (8, 128))
            out_cl_ref[...] = jnp.broadcast_to(cl, (8, 128))
            out_co_ref[...] = jnp.broadcast_to(co, (8, 128))


def kernel(batch_boxes, batch_classes, anchors, batch_gt, batch_num_objects):
    B, N, _ = batch_boxes.shape
    G = batch_gt.shape[1]
    CH = 4096 if N % 4096 == 0 else N
    K = N // CH
    IM = 2 if B % 2 == 0 else 1
    STEPS = B // IM

    f32 = jnp.float32
    anch_t = anchors.T.astype(f32)
    bx_t = jnp.swapaxes(batch_boxes, 1, 2).astype(f32)
    cl_t = jnp.swapaxes(batch_classes, 1, 2).astype(f32)
    valid = (jnp.arange(G)[None, :] < batch_num_objects[:, None]).astype(f32)
    degen = jnp.array([-100.0, -100.0, 1.0, 1.0], f32)
    gt_deg = jnp.where(valid[:, :, None] > 0.5, batch_gt.astype(f32),
                       degen[None, None, :])
    gt_ext = jnp.concatenate(
        [gt_deg, valid[:, :, None], jnp.zeros((B, G, 3), f32)], axis=2)

    body = functools.partial(
        _loss_kernel, n_steps=STEPS, n_chunks=K, n_im=IM, chunk=CH,
        n_total=N, n_gt=G)

    out_t, out_cl, out_co = pl.pallas_call(
        body,
        grid=(STEPS, K),
        in_specs=[
            pl.BlockSpec((4, CH), lambda i, k: (0, k)),
            pl.BlockSpec((IM, 4, CH), lambda i, k: (i, 0, k)),
            pl.BlockSpec((IM, 2, CH), lambda i, k: (i, 0, k)),
            pl.BlockSpec((IM, G, 8), lambda i, k: (i, 0, 0)),
        ],
        out_specs=[
            pl.BlockSpec((8, 128), lambda i, k: (0, 0)),
            pl.BlockSpec((8, 128), lambda i, k: (0, 0)),
            pl.BlockSpec((8, 128), lambda i, k: (0, 0)),
        ],
        out_shape=[
            jax.ShapeDtypeStruct((8, 128), f32),
            jax.ShapeDtypeStruct((8, 128), f32),
            jax.ShapeDtypeStruct((8, 128), f32),
        ],
        scratch_shapes=[
            pltpu.VMEM((8, 128), f32),
            pltpu.VMEM((IM * G, 128), f32),
            pltpu.VMEM((IM * G, 128), f32),
            pltpu.VMEM((IM * G, 128), f32),
            pltpu.VMEM((IM * G, 128), f32),
            pltpu.VMEM((IM * G, 128), f32),
        ],
        compiler_params=pltpu.CompilerParams(
            dimension_semantics=("arbitrary", "arbitrary")),
    )(anch_t, bx_t, cl_t, gt_ext)

    total = out_t[0, 0]
    cl = out_cl[0, 0]
    co = out_co[0, 0]
    return (total, cl, co)


# speedup vs baseline: 1.6717x; 1.0143x over previous
import functools

import jax
import jax.numpy as jnp
from jax.experimental import pallas as pl
from jax.experimental.pallas import tpu as pltpu


def _sum11(x):
    return jnp.sum(jnp.sum(x, axis=0, keepdims=True), axis=1, keepdims=True)


def _lanefold(x):
    parts = [x[:, j * 128:(j + 1) * 128] for j in range(x.shape[1] // 128)]
    while len(parts) > 1:
        h = len(parts) // 2
        merged = [parts[j] + parts[h + j] for j in range(h)]
        if len(parts) % 2:
            merged.append(parts[-1])
        parts = merged
    return parts[0]


def _sigmoids(d):
    ed = jnp.exp(-jnp.abs(d))
    r = 1.0 / (1.0 + ed)
    pos = d >= 0.0
    s1 = jnp.where(pos, r, ed * r)
    s0 = jnp.where(pos, ed * r, r)
    return s1, s0


def _loss_kernel(anch_ref, bx_ref, cl_ref, gt_ref,
                 out_t_ref, out_cl_ref, out_co_ref,
                 sacc, macc, bv, bidx, bl1, bd,
                 *, n_steps, n_chunks, n_im, chunk, n_total, n_gt):
    i = pl.program_id(0)
    k = pl.program_id(1)

    @pl.when(jnp.logical_and(i == 0, k == 0))
    def _():
        sacc[2 * n_im:2 * n_im + 2, :] = jnp.zeros((2, 128), jnp.float32)

    @pl.when(k == 0)
    def _():
        sacc[0:2 * n_im, :] = jnp.zeros((2 * n_im, 128), jnp.float32)
        macc[...] = jnp.zeros((n_im * n_gt, 128), jnp.float32)
        bv[...] = jnp.full((n_im * n_gt, 128), -1.0, jnp.float32)

    a_x0 = anch_ref[0:1, :]
    a_y0 = anch_ref[1:2, :]
    a_x1 = anch_ref[2:3, :]
    a_y1 = anch_ref[3:4, :]
    area_a = (a_x1 - a_x0) * (a_y1 - a_y0)
    lane = jax.lax.broadcasted_iota(jnp.int32, (n_gt, chunk), 1)
    bf = jnp.bfloat16

    for j in range(n_im):
        r0, r1 = j * n_gt, (j + 1) * n_gt

        b0 = bx_ref[j, 0:1, :]
        b1 = bx_ref[j, 1:2, :]
        b2 = bx_ref[j, 2:3, :]
        b3 = bx_ref[j, 3:4, :]
        d = cl_ref[j, 1:2, :] - cl_ref[j, 0:1, :]

        gtb = gt_ref[j]
        g_x0 = gtb[:, 0:1]
        g_y0 = gtb[:, 1:2]
        g_x1 = g_x0 + gtb[:, 2:3]
        g_y1 = g_y0 + gtb[:, 3:4]
        area_g = (g_x1 - g_x0) * (g_y1 - g_y0)

        wx = jnp.maximum(
            jnp.minimum(a_x1, g_x1) - jnp.maximum(a_x0, g_x0), 0.0)
        wy = jnp.maximum(
            jnp.minimum(a_y1, g_y1) - jnp.maximum(a_y0, g_y0), 0.0)
        inter = wx * wy
        union = area_a + area_g - inter
        iou = inter / union
        maskf = jnp.where(iou > 0.5, 1.0, 0.0)

        l1 = ((jnp.abs(b0.astype(bf) - g_x0.astype(bf))
               + jnp.abs(b1.astype(bf) - g_y0.astype(bf)))
              + (jnp.abs(b2.astype(bf) - g_x1.astype(bf))
                 + jnp.abs(b3.astype(bf) - g_y1.astype(bf)))
              ).astype(jnp.float32)

        macc[r0:r1, :] += _lanefold(maskf * l1)
        cnt_row = jnp.sum(maskf, axis=0, keepdims=True)
        sacc[n_im + j:n_im + j + 1, :] += _lanefold(cnt_row)

        anyrow = cnt_row > 0.5
        s1, s0 = _sigmoids(d)
        fl_pos = (-5.0) * s0 * s0 * jnp.log(s1)
        fl_neg = (-1.0) * s1 * s1 * jnp.log(s0)
        fl = jnp.where(anyrow, fl_pos, fl_neg)
        sacc[j:j + 1, :] += _lanefold(fl)

        cmax = jnp.max(iou, axis=1, keepdims=True)
        eqm = iou == cmax
        cand_lane = jnp.min(jnp.where(eqm, lane, jnp.int32(2**30)),
                            axis=1, keepdims=True)
        onehot = jnp.where(lane == cand_lane, 1.0, 0.0)
        ext_row = (jnp.clip(d, -1000.0, 1000.0)
                   + jnp.where(anyrow, 65536.0, 0.0))
        cand_l1 = jnp.sum(onehot * l1, axis=1, keepdims=True)
        cand_e = jnp.sum(onehot * ext_row, axis=1, keepdims=True)
        cand_idx = (cand_lane + chunk * k).astype(jnp.float32)

        upd = cmax > bv[r0:r1, 0:1]
        bv[r0:r1, :] = jnp.where(upd, cmax, bv[r0:r1, :])
        bidx[r0:r1, :] = jnp.where(upd, cand_idx, bidx[r0:r1, :])
        bl1[r0:r1, :] = jnp.where(upd, cand_l1, bl1[r0:r1, :])
        bd[r0:r1, :] = jnp.where(upd, cand_e, bd[r0:r1, :])

    @pl.when(k == n_chunks - 1)
    def _():
        for j in range(n_im):
            r0, r1 = j * n_gt, (j + 1) * n_gt
            validv = gt_ref[j][:, 4:5] > 0.5
            bv_v = bv[r0:r1, 0:1]
            bidx_v = bidx[r0:r1, 0:1]
            validf = jnp.where(validv, 1.0, 0.0)

            add_pair = jnp.where(
                jnp.logical_and(validv, bv_v <= 0.5), 1.0, 0.0)
            add_l1 = _sum11(add_pair * bl1[r0:r1, 0:1])
            add_cnt = _sum11(add_pair)
            csum_t = _sum11(macc[r0:r1, :]) + add_l1
            cnt_t = (jnp.sum(sacc[n_im + j:n_im + j + 1, :],
                             axis=1, keepdims=True) + add_cnt) * 4.0
            coord_img = csum_t / cnt_t

            ev = bd[r0:r1, 0:1]
            tentv = ev > 32768.0
            dv = ev - jnp.where(tentv, 65536.0, 0.0)
            s1, s0 = _sigmoids(dv)
            delta = ((-5.0) * s0 * s0 * jnp.log(s1)
                     + s1 * s1 * jnp.log(s0))

            iota_r = jax.lax.broadcasted_iota(jnp.int32, (n_gt, n_gt), 0)
            iota_c = jax.lax.broadcasted_iota(jnp.int32, (n_gt, n_gt), 1)
            diag = jnp.where(iota_r == iota_c, 1.0, 0.0)
            bidx_row = jnp.sum(diag * bidx_v, axis=0, keepdims=True)
            valid_row = jnp.sum(diag * validf, axis=0, keepdims=True)
            dupmat = jnp.where(
                jnp.logical_and(
                    jnp.logical_and(bidx_v == bidx_row, valid_row > 0.5),
                    iota_c < iota_r),
                1.0, 0.0)
            nondup = jnp.sum(dupmat, axis=1, keepdims=True) < 0.5
            fix = jnp.logical_and(
                jnp.logical_and(validv, jnp.logical_not(tentv)), nondup)
            corr = _sum11(jnp.where(fix, delta, 0.0))

            class_img = (jnp.sum(sacc[j:j + 1, :], axis=1, keepdims=True)
                         + corr)
            sacc[2 * n_im:2 * n_im + 1, 0:1] += (
                class_img * (1.0 / jnp.float32(n_total)))
            sacc[2 * n_im + 1:2 * n_im + 2, 0:1] += coord_img

        @pl.when(i == n_steps - 1)
        def _():
            inv_r = 1.0 / jnp.float32(n_steps * n_im)
            cl = sacc[2 * n_im:2 * n_im + 1, 0:1] * inv_r
            co = sacc[2 * n_im + 1:2 * n_im + 2, 0:1] * inv_r
            tot = cl + co
            out_t_ref[...] = jnp.broadcast_to(tot, (8, 128))
            out_cl_ref[...] = jnp.broadcast_to(cl, (8, 128))
            out_co_ref[...] = jnp.broadcast_to(co, (8, 128))


def kernel(batch_boxes, batch_classes, anchors, batch_gt, batch_num_objects):
    B, N, _ = batch_boxes.shape
    G = batch_gt.shape[1]
    CH = 4096 if N % 4096 == 0 else N
    K = N // CH
    IM = 4 if B % 4 == 0 else (2 if B % 2 == 0 else 1)
    STEPS = B // IM

    f32 = jnp.float32
    anch_t = anchors.T.astype(f32)
    bx_t = jnp.swapaxes(batch_boxes, 1, 2).astype(f32)
    cl_t = jnp.swapaxes(batch_classes, 1, 2).astype(f32)
    valid = (jnp.arange(G)[None, :] < batch_num_objects[:, None]).astype(f32)
    degen = jnp.array([-100.0, -100.0, 1.0, 1.0], f32)
    gt_deg = jnp.where(valid[:, :, None] > 0.5, batch_gt.astype(f32),
                       degen[None, None, :])
    gt_ext = jnp.concatenate(
        [gt_deg, valid[:, :, None], jnp.zeros((B, G, 3), f32)], axis=2)

    body = functools.partial(
        _loss_kernel, n_steps=STEPS, n_chunks=K, n_im=IM, chunk=CH,
        n_total=N, n_gt=G)

    out_t, out_cl, out_co = pl.pallas_call(
        body,
        grid=(STEPS, K),
        in_specs=[
            pl.BlockSpec((4, CH), lambda i, k: (0, k)),
            pl.BlockSpec((IM, 4, CH), lambda i, k: (i, 0, k)),
            pl.BlockSpec((IM, 2, CH), lambda i, k: (i, 0, k)),
            pl.BlockSpec((IM, G, 8), lambda i, k: (i, 0, 0)),
        ],
        out_specs=[
            pl.BlockSpec((8, 128), lambda i, k: (0, 0)),
            pl.BlockSpec((8, 128), lambda i, k: (0, 0)),
            pl.BlockSpec((8, 128), lambda i, k: (0, 0)),
        ],
        out_shape=[
            jax.ShapeDtypeStruct((8, 128), f32),
            jax.ShapeDtypeStruct((8, 128), f32),
            jax.ShapeDtypeStruct((8, 128), f32),
        ],
        scratch_shapes=[
            pltpu.VMEM((16, 128), f32),
            pltpu.VMEM((IM * G, 128), f32),
            pltpu.VMEM((IM * G, 128), f32),
            pltpu.VMEM((IM * G, 128), f32),
            pltpu.VMEM((IM * G, 128), f32),
            pltpu.VMEM((IM * G, 128), f32),
        ],
        compiler_params=pltpu.CompilerParams(
            dimension_semantics=("arbitrary", "arbitrary")),
    )(anch_t, bx_t, cl_t, gt_ext)

    total = out_t[0, 0]
    cl = out_cl[0, 0]
    co = out_co[0, 0]
    return (total, cl, co)
